# Initial kernel scaffold; baseline (speedup 1.0000x reference)
import jax
import jax.numpy as jnp
from jax.experimental import pallas as pl


def kernel(grid, mask, sub_enc_w0, sub_enc_b0, sub_enc_w1, sub_enc_b1, sub_cls_w0, sub_cls_b0, sub_cls_w1, sub_cls_b1, add_w0, add_b0, add_w1, add_b1, add_w2, add_b2):
    raise NotImplementedError("write your pallas kernel here")


# speedup vs baseline: 3.1261x; 3.1261x over previous
import jax
import jax.numpy as jnp
from jax.experimental import pallas as pl

B, H, W = 4, 32, 64
CHUNK = 4
NQ = W // CHUNK
NPAD = 48
MAX_VALUE = 50.0

_DN = (((1,), (1,)), ((), ()))


def _counts_kernel(grid_ref, mask_ref, e0_ref, e0b_ref, e1_ref, e1b_ref,
                   c0_ref, c0b_ref, c1_ref, c1b_ref,
                   counts_ref, lut_ref):
    f32 = jnp.float32
    ext = jnp.where(mask_ref[...] > 0, grid_ref[...], 0.0)
    bits = (ext > 0).astype(f32).reshape(B * H, W)

    wi = jax.lax.broadcasted_iota(jnp.int32, (W, NQ), 0)
    qi = jax.lax.broadcasted_iota(jnp.int32, (W, NQ), 1)
    sel = jnp.where((wi // CHUNK) == qi,
                    jax.lax.shift_left(1, wi % CHUNK), 0).astype(f32)
    pattern = jax.lax.dot_general(bits, sel, (((1,), (0,)), ((), ())),
                                  preferred_element_type=f32)
    patt_i = pattern.astype(jnp.int32)

    ri = jax.lax.broadcasted_iota(jnp.int32, (16 * CHUNK, 1), 0)
    bitcol = (jax.lax.shift_right_logical(ri // CHUNK, ri % CHUNK) & 1
              ).astype(f32)
    poscol = (ri % CHUNK).astype(f32) / CHUNK
    x = jnp.concatenate([bitcol, poscol], axis=1)
    h1 = jax.nn.relu(jax.lax.dot_general(x, e0_ref[...], _DN,
                                         preferred_element_type=f32)
                     + e0b_ref[...])
    h2 = jax.nn.relu(jax.lax.dot_general(h1, e1_ref[...], _DN,
                                         preferred_element_type=f32)
                     + e1b_ref[...])
    pi = jax.lax.broadcasted_iota(jnp.int32, (16, 16 * CHUNK), 0)
    rj = jax.lax.broadcasted_iota(jnp.int32, (16, 16 * CHUNK), 1)
    mmat = jnp.where((rj // CHUNK) == pi, 0.25, 0.0).astype(f32)
    pooled = jax.lax.dot_general(mmat, h2, (((1,), (0,)), ((), ())),
                                 preferred_element_type=f32)
    cc = jax.nn.relu(jax.lax.dot_general(pooled, c0_ref[...], _DN,
                                         preferred_element_type=f32)
                     + c0b_ref[...])
    c2 = jax.nn.relu(jnp.sum(cc * c1_ref[...], axis=1, keepdims=True)
                     + c1b_ref[0, 0])
    lut = jnp.round(c2)
    lut_ref[...] = lut

    counts = jnp.zeros((B * H, NQ), dtype=f32)
    for p in range(16):
        counts = jnp.where(patt_i == p, lut[p, 0], counts)
    counts_ref[...] = counts


def _scan_kernel(counts_ref, lut_ref, a0_ref, a0b_ref, a1_ref, a1b_ref,
                 a2_ref, a2b_ref, out_ref):
    f32 = jnp.float32
    a0 = a0_ref[...]
    a0b = a0b_ref[...]
    a1 = a1_ref[...]
    a1b = a1b_ref[...]
    a2 = a2_ref[...]
    a2b = a2b_ref[0, 0]
    xc0 = jnp.full((B, 1), lut_ref[0, 0] / MAX_VALUE, dtype=f32)

    def step(r, xc):
        x = jnp.concatenate([r / MAX_VALUE, xc], axis=1)
        a = jax.nn.relu(jax.lax.dot_general(x, a0, _DN,
                                            preferred_element_type=f32) + a0b)
        a = jax.nn.relu(jax.lax.dot_general(a, a1, _DN,
                                            preferred_element_type=f32) + a1b)
        o = (jnp.sum(a * a2, axis=1, keepdims=True) + a2b) * MAX_VALUE
        return jnp.round(o)

    def h_body(h, r):
        blk = counts_ref[pl.ds(h * NQ * B, NQ * B), :]
        for q in range(NQ):
            r = step(r, blk[q * B:(q + 1) * B, :] / MAX_VALUE)
        return jax.lax.fori_loop(0, NPAD, lambda i, rr: step(rr, xc0), r)

    out_ref[...] = jax.lax.fori_loop(0, H, h_body, jnp.zeros((B, 1),
                                                             dtype=f32))


def kernel(grid, mask, sub_enc_w0, sub_enc_b0, sub_enc_w1, sub_enc_b1,
           sub_cls_w0, sub_cls_b0, sub_cls_w1, sub_cls_b1,
           add_w0, add_b0, add_w1, add_b1, add_w2, add_b2):
    f32 = jnp.float32
    counts, lut = pl.pallas_call(
        _counts_kernel,
        out_shape=[jax.ShapeDtypeStruct((B * H, NQ), f32),
                   jax.ShapeDtypeStruct((16, 1), f32)],
    )(grid, mask,
      sub_enc_w0, sub_enc_b0.reshape(1, 64),
      sub_enc_w1, sub_enc_b1.reshape(1, 64),
      sub_cls_w0, sub_cls_b0.reshape(1, 32),
      sub_cls_w1, sub_cls_b1.reshape(1, 1))
    counts_scan = counts.reshape(B, H, NQ).transpose(1, 2, 0).reshape(
        H * NQ * B, 1)
    total = pl.pallas_call(
        _scan_kernel,
        out_shape=jax.ShapeDtypeStruct((B, 1), f32),
    )(counts_scan, lut,
      add_w0, add_b0.reshape(1, 128),
      add_w1, add_b1.reshape(1, 128),
      add_w2, add_b2.reshape(1, 1))
    return total.reshape(B)


# speedup vs baseline: 11.2276x; 3.5916x over previous
import jax
import jax.numpy as jnp
from jax.experimental import pallas as pl

B, H, W = 4, 32, 64
CHUNK = 4
NQ = W // CHUNK
NPAD = 48
MAX_VALUE = 50.0

_DN = (((1,), (1,)), ((), ()))


def _counts_kernel(grid_ref, mask_ref, e0_ref, e0b_ref, e1_ref, e1b_ref,
                   c0_ref, c0b_ref, c1_ref, c1b_ref,
                   counts_ref, lut_ref):
    f32 = jnp.float32
    ext = jnp.where(mask_ref[...] > 0, grid_ref[...], 0.0)
    bits = (ext > 0).astype(f32).reshape(B * H, W)

    wi = jax.lax.broadcasted_iota(jnp.int32, (W, NQ), 0)
    qi = jax.lax.broadcasted_iota(jnp.int32, (W, NQ), 1)
    sel = jnp.where((wi // CHUNK) == qi,
                    jax.lax.shift_left(1, wi % CHUNK), 0).astype(f32)
    pattern = jax.lax.dot_general(bits, sel, (((1,), (0,)), ((), ())),
                                  preferred_element_type=f32)
    patt_i = pattern.astype(jnp.int32)

    ri = jax.lax.broadcasted_iota(jnp.int32, (16 * CHUNK, 1), 0)
    bitcol = (jax.lax.shift_right_logical(ri // CHUNK, ri % CHUNK) & 1
              ).astype(f32)
    poscol = (ri % CHUNK).astype(f32) / CHUNK
    x = jnp.concatenate([bitcol, poscol], axis=1)
    h1 = jax.nn.relu(jax.lax.dot_general(x, e0_ref[...], _DN,
                                         preferred_element_type=f32)
                     + e0b_ref[...])
    h2 = jax.nn.relu(jax.lax.dot_general(h1, e1_ref[...], _DN,
                                         preferred_element_type=f32)
                     + e1b_ref[...])
    pi = jax.lax.broadcasted_iota(jnp.int32, (16, 16 * CHUNK), 0)
    rj = jax.lax.broadcasted_iota(jnp.int32, (16, 16 * CHUNK), 1)
    mmat = jnp.where((rj // CHUNK) == pi, 0.25, 0.0).astype(f32)
    pooled = jax.lax.dot_general(mmat, h2, (((1,), (0,)), ((), ())),
                                 preferred_element_type=f32)
    cc = jax.nn.relu(jax.lax.dot_general(pooled, c0_ref[...], _DN,
                                         preferred_element_type=f32)
                     + c0b_ref[...])
    c2 = jax.nn.relu(jnp.sum(cc * c1_ref[...], axis=1, keepdims=True)
                     + c1b_ref[0, 0])
    lut = jnp.round(c2)
    lut_ref[...] = lut

    counts = jnp.zeros((B * H, NQ), dtype=f32)
    for p in range(16):
        counts = jnp.where(patt_i == p, lut[p, 0], counts)
    counts_ref[...] = counts


def _scan_kernel(counts_ref, lut_ref, a0_ref, a0b_ref, a1_ref, a1b_ref,
                 a2_ref, a2b_ref, out_ref):
    f32 = jnp.float32
    a0 = a0_ref[...]
    a0b = a0b_ref[...]
    a1 = a1_ref[...]
    a1b = a1b_ref[...]
    a2 = a2_ref[...]
    a2b = a2b_ref[0, 0]
    xc0 = jnp.full((B, 1), lut_ref[0, 0] / MAX_VALUE, dtype=f32)

    def step(r, xc):
        x = jnp.concatenate([r / MAX_VALUE, xc], axis=1)
        a = jax.nn.relu(jax.lax.dot_general(x, a0, _DN,
                                            preferred_element_type=f32) + a0b)
        a = jax.nn.relu(jax.lax.dot_general(a, a1, _DN,
                                            preferred_element_type=f32) + a1b)
        o = (jnp.sum(a * a2, axis=1, keepdims=True) + a2b) * MAX_VALUE
        return jnp.round(o)

    def h_body(h, r):
        blk = counts_ref[pl.ds(h * NQ * B, NQ * B), :]
        for q in range(NQ):
            r = step(r, blk[q * B:(q + 1) * B, :] / MAX_VALUE)

        def pad_cond(carry):
            i, _, moved = carry
            return jnp.logical_and(i < NPAD, moved)

        def pad_body(carry):
            i, rr, _ = carry
            r2 = step(rr, xc0)
            return i + 1, r2, jnp.any(r2 != rr)

        _, r, _ = jax.lax.while_loop(pad_cond, pad_body,
                                     (jnp.int32(0), r, True))
        return r

    out_ref[...] = jax.lax.fori_loop(0, H, h_body, jnp.zeros((B, 1),
                                                             dtype=f32))


def kernel(grid, mask, sub_enc_w0, sub_enc_b0, sub_enc_w1, sub_enc_b1,
           sub_cls_w0, sub_cls_b0, sub_cls_w1, sub_cls_b1,
           add_w0, add_b0, add_w1, add_b1, add_w2, add_b2):
    f32 = jnp.float32
    counts, lut = pl.pallas_call(
        _counts_kernel,
        out_shape=[jax.ShapeDtypeStruct((B * H, NQ), f32),
                   jax.ShapeDtypeStruct((16, 1), f32)],
    )(grid, mask,
      sub_enc_w0, sub_enc_b0.reshape(1, 64),
      sub_enc_w1, sub_enc_b1.reshape(1, 64),
      sub_cls_w0, sub_cls_b0.reshape(1, 32),
      sub_cls_w1, sub_cls_b1.reshape(1, 1))
    counts_scan = counts.reshape(B, H, NQ).transpose(1, 2, 0).reshape(
        H * NQ * B, 1)
    total = pl.pallas_call(
        _scan_kernel,
        out_shape=jax.ShapeDtypeStruct((B, 1), f32),
    )(counts_scan, lut,
      add_w0, add_b0.reshape(1, 128),
      add_w1, add_b1.reshape(1, 128),
      add_w2, add_b2.reshape(1, 1))
    return total.reshape(B)


# speedup vs baseline: 15.1119x; 1.3460x over previous
import jax
import jax.numpy as jnp
from jax.experimental import pallas as pl
from jax.experimental.pallas import tpu as pltpu

B, H, W = 4, 32, 64
CHUNK = 4
NQ = W // CHUNK
NPAD = 48
MAX_VALUE = 50.0

_DN = (((1,), (1,)), ((), ()))


def _counts_kernel(grid_ref, mask_ref, e0_ref, e0b_ref, e1_ref, e1b_ref,
                   c0_ref, c0b_ref, c1_ref, c1b_ref,
                   counts_ref, lut_ref):
    f32 = jnp.float32
    ext = jnp.where(mask_ref[...] > 0, grid_ref[...], 0.0)
    bits = (ext > 0).astype(f32).reshape(B * H, W)

    wi = jax.lax.broadcasted_iota(jnp.int32, (W, NQ), 0)
    qi = jax.lax.broadcasted_iota(jnp.int32, (W, NQ), 1)
    sel = jnp.where((wi // CHUNK) == qi,
                    jax.lax.shift_left(1, wi % CHUNK), 0).astype(f32)
    pattern = jax.lax.dot_general(bits, sel, (((1,), (0,)), ((), ())),
                                  preferred_element_type=f32)
    patt_i = pattern.astype(jnp.int32)

    ri = jax.lax.broadcasted_iota(jnp.int32, (16 * CHUNK, 1), 0)
    bitcol = (jax.lax.shift_right_logical(ri // CHUNK, ri % CHUNK) & 1
              ).astype(f32)
    poscol = (ri % CHUNK).astype(f32) / CHUNK
    x = jnp.concatenate([bitcol, poscol], axis=1)
    h1 = jax.nn.relu(jax.lax.dot_general(x, e0_ref[...], _DN,
                                         preferred_element_type=f32)
                     + e0b_ref[...])
    h2 = jax.nn.relu(jax.lax.dot_general(h1, e1_ref[...], _DN,
                                         preferred_element_type=f32)
                     + e1b_ref[...])
    pi = jax.lax.broadcasted_iota(jnp.int32, (16, 16 * CHUNK), 0)
    rj = jax.lax.broadcasted_iota(jnp.int32, (16, 16 * CHUNK), 1)
    mmat = jnp.where((rj // CHUNK) == pi, 0.25, 0.0).astype(f32)
    pooled = jax.lax.dot_general(mmat, h2, (((1,), (0,)), ((), ())),
                                 preferred_element_type=f32)
    cc = jax.nn.relu(jax.lax.dot_general(pooled, c0_ref[...], _DN,
                                         preferred_element_type=f32)
                     + c0b_ref[...])
    c2 = jax.nn.relu(jnp.sum(cc * c1_ref[...], axis=1, keepdims=True)
                     + c1b_ref[0, 0])
    lut = jnp.round(c2)
    lut_ref[...] = lut

    counts = jnp.zeros((B * H, NQ), dtype=f32)
    for p in range(16):
        counts = jnp.where(patt_i == p, lut[p, 0], counts)
    counts_ref[...] = counts


def _scan_kernel(counts_ref, lut_ref, u_ref, v_ref, a0b_ref, a1_ref,
                 a1b_ref, a2r_ref, a2b_ref, out_ref, cv_ref):
    f32 = jnp.float32
    u = u_ref[...]
    a0b = a0b_ref[...]
    a1 = a1_ref[...]
    a1b = a1b_ref[...]
    a2r = a2r_ref[...]
    a2b = a2b_ref[0, 0]

    cv_ref[...] = jax.lax.dot_general(
        counts_ref[...], v_ref[...], (((1,), (0,)), ((), ())),
        preferred_element_type=f32) + a0b
    cv0 = lut_ref[0, 0] * v_ref[...] + a0b

    def step(R, cv):
        h1 = jax.nn.relu(R * u + cv)
        a = jax.nn.relu(jax.lax.dot_general(h1, a1, _DN,
                                            preferred_element_type=f32) + a1b)
        Ob = jax.lax.dot_general(a, a2r, _DN, preferred_element_type=f32)
        return jnp.round((Ob + a2b) * MAX_VALUE)

    def h_body(h, R):
        base = h * (NQ * B)
        for q in range(NQ):
            R = step(R, cv_ref[pl.ds(base + q * B, B), :])

        def pad_cond(carry):
            i, _, moved = carry
            return jnp.logical_and(i < NPAD, moved)

        def pad_body(carry):
            i, rr, _ = carry
            r2 = step(rr, cv0)
            return i + 1, r2, jnp.any(r2 != rr)

        _, R, _ = jax.lax.while_loop(pad_cond, pad_body,
                                     (jnp.int32(0), R, True))
        return R

    out_ref[...] = jax.lax.fori_loop(0, H, h_body,
                                     jnp.zeros((B, 128), dtype=f32))[:, 0:1]


def kernel(grid, mask, sub_enc_w0, sub_enc_b0, sub_enc_w1, sub_enc_b1,
           sub_cls_w0, sub_cls_b0, sub_cls_w1, sub_cls_b1,
           add_w0, add_b0, add_w1, add_b1, add_w2, add_b2):
    f32 = jnp.float32
    counts, lut = pl.pallas_call(
        _counts_kernel,
        out_shape=[jax.ShapeDtypeStruct((B * H, NQ), f32),
                   jax.ShapeDtypeStruct((16, 1), f32)],
    )(grid, mask,
      sub_enc_w0, sub_enc_b0.reshape(1, 64),
      sub_enc_w1, sub_enc_b1.reshape(1, 64),
      sub_cls_w0, sub_cls_b0.reshape(1, 32),
      sub_cls_w1, sub_cls_b1.reshape(1, 1))
    counts_scan = counts.reshape(B, H, NQ).transpose(1, 2, 0).reshape(
        H * NQ * B, 1)
    total = pl.pallas_call(
        _scan_kernel,
        out_shape=jax.ShapeDtypeStruct((B, 1), f32),
        scratch_shapes=[pltpu.VMEM((H * NQ * B, 128), f32)],
    )(counts_scan, lut,
      (add_w0[:, 0] / MAX_VALUE).reshape(1, 128),
      (add_w0[:, 1] / MAX_VALUE).reshape(1, 128),
      add_b0.reshape(1, 128),
      add_w1, add_b1.reshape(1, 128),
      jnp.broadcast_to(add_w2, (128, 128)),
      add_b2.reshape(1, 1))
    return total.reshape(B)


# speedup vs baseline: 95.7658x; 6.3371x over previous
import jax
import jax.numpy as jnp
from jax.experimental import pallas as pl
from jax.experimental.pallas import tpu as pltpu

B, H, W = 4, 32, 64
CHUNK = 4
NQ = W // CHUNK
NPAD = 48
MAX_VALUE = 50.0

_DN = (((1,), (1,)), ((), ()))
_DNK = (((1,), (0,)), ((), ()))


def _counts_kernel(grid_ref, mask_ref, e0_ref, e0b_ref, e1_ref, e1b_ref,
                   c0_ref, c0b_ref, c1_ref, c1b_ref,
                   counts_ref, lut_ref):
    f32 = jnp.float32
    ext = jnp.where(mask_ref[...] > 0, grid_ref[...], 0.0)
    bits = (ext > 0).astype(f32).reshape(B * H, W)

    wi = jax.lax.broadcasted_iota(jnp.int32, (W, NQ), 0)
    qi = jax.lax.broadcasted_iota(jnp.int32, (W, NQ), 1)
    sel = jnp.where((wi // CHUNK) == qi,
                    jax.lax.shift_left(1, wi % CHUNK), 0).astype(f32)
    pattern = jax.lax.dot_general(bits, sel, _DNK,
                                  preferred_element_type=f32)
    patt_i = pattern.astype(jnp.int32)

    ri = jax.lax.broadcasted_iota(jnp.int32, (16 * CHUNK, 1), 0)
    bitcol = (jax.lax.shift_right_logical(ri // CHUNK, ri % CHUNK) & 1
              ).astype(f32)
    poscol = (ri % CHUNK).astype(f32) / CHUNK
    x = jnp.concatenate([bitcol, poscol], axis=1)
    h1 = jax.nn.relu(jax.lax.dot_general(x, e0_ref[...], _DN,
                                         preferred_element_type=f32)
                     + e0b_ref[...])
    h2 = jax.nn.relu(jax.lax.dot_general(h1, e1_ref[...], _DN,
                                         preferred_element_type=f32)
                     + e1b_ref[...])
    pi = jax.lax.broadcasted_iota(jnp.int32, (16, 16 * CHUNK), 0)
    rj = jax.lax.broadcasted_iota(jnp.int32, (16, 16 * CHUNK), 1)
    mmat = jnp.where((rj // CHUNK) == pi, 0.25, 0.0).astype(f32)
    pooled = jax.lax.dot_general(mmat, h2, _DNK,
                                 preferred_element_type=f32)
    cc = jax.nn.relu(jax.lax.dot_general(pooled, c0_ref[...], _DN,
                                         preferred_element_type=f32)
                     + c0b_ref[...])
    c2 = jax.nn.relu(jnp.sum(cc * c1_ref[...], axis=1, keepdims=True)
                     + c1b_ref[0, 0])
    lut = jnp.round(c2)
    lut_ref[...] = lut

    counts = jnp.zeros((B * H, NQ), dtype=f32)
    for p in range(16):
        counts = jnp.where(patt_i == p, lut[p, 0], counts)
    counts_ref[...] = counts


def _scan_kernel(counts_ref, lut_ref, u_ref, v_ref, a0b_ref, a1_ref,
                 a1b_ref, a2r_ref, a2b_ref, out_ref, cv_ref, e_ref):
    f32 = jnp.float32
    u = u_ref[...]
    a0b = a0b_ref[...]
    a1 = a1_ref[...]
    a1b = a1b_ref[...]
    a2r = a2r_ref[...]
    a2b = a2b_ref[0, 0]

    cv_ref[...] = jax.lax.dot_general(
        counts_ref[...], v_ref[...], _DNK,
        preferred_element_type=f32) + a0b
    cv0 = lut_ref[0, 0] * v_ref[...] + a0b

    def mlp(R, cv):
        h1 = jax.nn.relu(R * u + cv)
        a = jax.nn.relu(jax.lax.dot_general(h1, a1, _DNK,
                                            preferred_element_type=f32) + a1b)
        Ob = jax.lax.dot_general(a, a2r, _DNK, preferred_element_type=f32)
        return jnp.round((Ob + a2b) * MAX_VALUE)

    def padded_run(R):
        def cond(c):
            return jnp.logical_and(c[0] < NPAD, c[2])

        def body(c):
            i, rr, _ = c
            r2 = mlp(rr, cv0)
            return i + 1, r2, jnp.any(r2 != rr)

        return jax.lax.while_loop(cond, body, (jnp.int32(0), R, True))[1]

    rstar = padded_run(jnp.zeros((1, 128), dtype=f32))

    sub = jax.lax.broadcasted_iota(jnp.int32, (H * B, 128), 0)
    R = jnp.where(sub < B, 0.0, rstar)
    for q in range(NQ):
        R = mlp(R, cv_ref[pl.ds(q * H * B, H * B), :])
    e_ref[...] = padded_run(R)

    def seq_row(h, st):
        for q in range(NQ):
            st = mlp(st, cv_ref[pl.ds(q * H * B + h * B, B), :])
        return padded_run(st)

    def stitch(h, st):
        eh = e_ref[pl.ds(h * B, B), :]
        return jax.lax.cond(jnp.all(st == rstar),
                            lambda s: eh, lambda s: seq_row(h, s), st)

    state = jax.lax.fori_loop(1, H, stitch, e_ref[0:B, :])
    out_ref[...] = state[:, 0:1]


def kernel(grid, mask, sub_enc_w0, sub_enc_b0, sub_enc_w1, sub_enc_b1,
           sub_cls_w0, sub_cls_b0, sub_cls_w1, sub_cls_b1,
           add_w0, add_b0, add_w1, add_b1, add_w2, add_b2):
    f32 = jnp.float32
    counts, lut = pl.pallas_call(
        _counts_kernel,
        out_shape=[jax.ShapeDtypeStruct((B * H, NQ), f32),
                   jax.ShapeDtypeStruct((16, 1), f32)],
    )(grid, mask,
      sub_enc_w0, sub_enc_b0.reshape(1, 64),
      sub_enc_w1, sub_enc_b1.reshape(1, 64),
      sub_cls_w0, sub_cls_b0.reshape(1, 32),
      sub_cls_w1, sub_cls_b1.reshape(1, 1))
    counts_scan = counts.reshape(B, H, NQ).transpose(2, 1, 0).reshape(
        NQ * H * B, 1)
    total = pl.pallas_call(
        _scan_kernel,
        out_shape=jax.ShapeDtypeStruct((B, 1), f32),
        scratch_shapes=[pltpu.VMEM((NQ * H * B, 128), f32),
                        pltpu.VMEM((H * B, 128), f32)],
    )(counts_scan, lut,
      (add_w0[:, 0] / MAX_VALUE).reshape(1, 128),
      (add_w0[:, 1] / MAX_VALUE).reshape(1, 128),
      add_b0.reshape(1, 128),
      add_w1.T, add_b1.reshape(1, 128),
      jnp.broadcast_to(add_w2.reshape(128, 1), (128, 128)),
      add_b2.reshape(1, 1))
    return total.reshape(B)
